# baseline (speedup 1.0000x reference)
import functools

import jax
import jax.numpy as jnp
from jax.experimental import pallas as pl
from jax.experimental.pallas import tpu as pltpu

_B = 128
_C = 384
_S = 196
_D1 = 1024
_THRES = 0.3
_EPS = 1e-6
_BB = 8
_G = 6272
_NG = 12


def _m2_body(x_ref, m_ref):
    x = x_ref[...]
    m_ref[...] = jnp.sum(x * x, axis=1) * (1.0 / _C)


def _m2_map(f2):
    return pl.pallas_call(
        _m2_body,
        grid=(_B // _BB,),
        in_specs=[pl.BlockSpec((_BB, _C, _S), lambda i: (i, 0, 0))],
        out_specs=pl.BlockSpec((_BB, _S), lambda i: (i, 0)),
        out_shape=jax.ShapeDtypeStruct((_B, _S), jnp.float32),
    )(f2)


def _stats_body(x_ref, m2_ref, out_ref, z_ref):
    j = pl.program_id(1)
    x = x_ref[...]

    @pl.when(j == 0)
    def _():
        z_ref[...] = x * x

    @pl.when(j > 0)
    def _():
        z_ref[...] = z_ref[...] + x * x

    @pl.when(j == _NG - 1)
    def _():
        z = z_ref[...]
        m1 = jnp.zeros((_BB, _S), jnp.float32)
        for k in range(32):
            m1 = m1 + z[:, k * _S:(k + 1) * _S]
        m1 = m1 * (1.0 / _C)
        m2 = m2_ref[...]
        d = m1 - m2
        mse_b = jnp.sum(d * d, axis=1, keepdims=True)
        mx1 = jnp.max(m1, axis=1, keepdims=True)
        mn1 = jnp.min(m1, axis=1, keepdims=True)
        mx2 = jnp.max(m2, axis=1, keepdims=True)
        mn2 = jnp.min(m2, axis=1, keepdims=True)
        zz = jnp.zeros((_BB, 3), jnp.float32)
        out_ref[...] = jnp.concatenate([mse_b, mx1, mn1, mx2, mn2, zz],
                                       axis=1)


def _stats(f1flat, m2part):
    return pl.pallas_call(
        _stats_body,
        grid=(_B // _BB, _NG),
        in_specs=[
            pl.BlockSpec((_BB, _G), lambda i, j: (i, j)),
            pl.BlockSpec((_BB, _S), lambda i, j: (i, 0)),
        ],
        out_specs=pl.BlockSpec((_BB, 8), lambda i, j: (i, 0)),
        out_shape=jax.ShapeDtypeStruct((_B, 8), jnp.float32),
        scratch_shapes=[pltpu.VMEM((_BB, _G), jnp.float32)],
        compiler_params=pltpu.CompilerParams(
            dimension_semantics=("arbitrary", "arbitrary"),
        ),
    )(f1flat, m2part)


def _fin_body(part_ref, a_ref, b_ref, q1c_ref, q2c_ref, w_ref,
              loss_ref, wl_ref):
    q1c = q1c_ref[...]
    q2c = q2c_ref[...]
    qm = jnp.abs(q1c - q2c)
    mean_q = jnp.sum(qm) * (1.0 / _B)
    margin_upper = 100.0 - (100.0 - mean_q) * _THRES
    margin_lower = mean_q * _THRES
    maskq = (qm < margin_lower) | (qm > margin_upper)
    q1z = jnp.where(maskq, 0.0, q1c)
    q2z = jnp.where(maskq, 0.0, q2c)
    f1h = q1z > q2z
    act = q1z != q2z
    count = jnp.sum(act.astype(jnp.float32))
    sum_q1 = jnp.sum(q1z)

    part = part_ref[...]
    mse_b = part[:, 0:1]
    mx1 = part[:, 1:2]
    mn1 = part[:, 2:3]
    mx2 = part[:, 3:4]
    mn2 = part[:, 4:5]
    fs_mx = jnp.where(f1h, mx2, mx1)
    fs_mn = jnp.where(f1h, mn2, mn1)
    mse_sum = jnp.sum(jnp.where(act, mse_b, 0.0))
    fs_max = jnp.max(jnp.where(act, fs_mx, -jnp.inf))
    fs_min = jnp.min(jnp.where(act, fs_mn, jnp.inf))

    a = a_ref[...]
    b = b_ref[...]
    dot = jnp.sum(a * b, axis=1, keepdims=True)
    na = jnp.sqrt(jnp.sum(a * a, axis=1, keepdims=True))
    nb = jnp.sqrt(jnp.sum(b * b, axis=1, keepdims=True))
    denom = jnp.maximum(na, _EPS) * jnp.maximum(nb, _EPS)
    cd = 1.0 - dot / denom
    cos_sum = jnp.sum(jnp.where(act, cd, 0.0))

    mse_loss = mse_sum / (count * jnp.float32(_S))
    ampify = 2.0 / (fs_max - fs_min)
    loss0 = ampify * mse_loss
    loss1 = cos_sum / count

    w0 = w_ref[0]
    w1 = w_ref[1]
    wl0 = loss0 * w0
    wl1 = loss1 * w1
    loss_all = wl0 + wl1

    zero_case = sum_q1 == 0.0
    loss_all = jnp.where(zero_case, 0.0, loss_all)
    wl0 = jnp.where(zero_case, 0.0, wl0)
    wl1 = jnp.where(zero_case, 0.0, wl1)

    loss_ref[...] = jnp.full((1, 1), loss_all)
    wl_ref[...] = jnp.concatenate(
        [jnp.full((1, 1), wl0), jnp.full((1, 1), wl1)], axis=1)


def _finalize(part, f1l1, f2l1, q1c, q2c, w):
    out = pl.pallas_call(
        _fin_body,
        in_specs=[
            pl.BlockSpec((_B, 8), lambda: (0, 0)),
            pl.BlockSpec((_B, _D1), lambda: (0, 0)),
            pl.BlockSpec((_B, _D1), lambda: (0, 0)),
            pl.BlockSpec((_B, 1), lambda: (0, 0)),
            pl.BlockSpec((_B, 1), lambda: (0, 0)),
            pl.BlockSpec(memory_space=pltpu.SMEM),
        ],
        out_specs=[
            pl.BlockSpec((1, 1), lambda: (0, 0)),
            pl.BlockSpec((1, 2), lambda: (0, 0)),
        ],
        out_shape=[
            jax.ShapeDtypeStruct((1, 1), jnp.float32),
            jax.ShapeDtypeStruct((1, 2), jnp.float32),
        ],
    )(part, f1l1, f2l1, q1c, q2c, w)
    return out[0].reshape(()), out[1].reshape(2)


@jax.jit
def _qsd_loss(f1l0, f1l1, f2l0, f2l1, q1, q2, w):
    f2 = f2l0.reshape(_B, _C, _S)
    f1flat = f1l0.reshape(_B, _C * _S)
    m2part = _m2_map(f2)
    part = _stats(f1flat, m2part)
    return _finalize(part, f1l1, f2l1,
                     q1.reshape(_B, 1), q2.reshape(_B, 1), w)


def kernel(features_1_level0, features_1_level1, features_2_level0,
           features_2_level1, quality_1, quality_2, weights):
    return _qsd_loss(features_1_level0, features_1_level1,
                     features_2_level0, features_2_level1,
                     quality_1, quality_2, weights)


# speedup vs baseline: 1.9748x; 1.9748x over previous
import functools

import jax
import jax.numpy as jnp
from jax import lax
from jax.experimental import pallas as pl
from jax.experimental.pallas import tpu as pltpu
from jax.experimental.pallas import tpu_sc as plsc

_B = 128
_C = 384
_S = 196
_D1 = 1024
_THRES = 0.3
_EPS = 1e-6
_BB = 8
_NCHUNK = 13


def _sc_body(f1_hbm, out_hbm, xbuf, row, sem):
    del sem
    ncores = 2
    wid = lax.axis_index("s") * ncores + lax.axis_index("c")
    lanes = lax.iota(jnp.int32, 16)
    tail_mask = lanes >= 12

    zero16 = jnp.zeros((16,), jnp.float32)
    for t in range(13, 16):
        row[pl.ds(16 * t, 16)] = zero16

    for kk in range(_B // 32):
        b = wid * (_B // 32) + kk
        pltpu.sync_copy(f1_hbm.at[b], xbuf)

        def cbody(c, accs):
            new = []
            for g in range(12):
                v = xbuf[c, pl.ds(16 * g, 16)]
                new.append(accs[g] + v * v)
            v = xbuf[c, pl.ds(180, 16)]
            new.append(accs[12] + jnp.where(tail_mask, v * v, 0.0))
            return tuple(new)

        accs = lax.fori_loop(0, _C, cbody,
                             tuple(zero16 for _ in range(_NCHUNK)))
        for g in range(_NCHUNK):
            row[pl.ds(16 * g, 16)] = accs[g]
        pltpu.sync_copy(row, out_hbm.at[b])


def _sc_m1_map(f1):
    mesh = plsc.VectorSubcoreMesh(core_axis_name="c", subcore_axis_name="s")
    return pl.kernel(
        _sc_body,
        out_type=jax.ShapeDtypeStruct((_B, 256), jnp.float32),
        mesh=mesh,
        scratch_types=[
            pltpu.VMEM((_C, _S), jnp.float32),
            pltpu.VMEM((256,), jnp.float32),
            pltpu.SemaphoreType.DMA,
        ],
        compiler_params=pltpu.CompilerParams(needs_layout_passes=False),
    )(f1)


def _m2_body(x_ref, m_ref):
    x = x_ref[...]
    m_ref[...] = jnp.sum(x * x, axis=1) * (1.0 / _C)


def _m2_map(f2):
    return pl.pallas_call(
        _m2_body,
        grid=(_B // _BB,),
        in_specs=[pl.BlockSpec((_BB, _C, _S), lambda i: (i, 0, 0))],
        out_specs=pl.BlockSpec((_BB, _S), lambda i: (i, 0)),
        out_shape=jax.ShapeDtypeStruct((_B, _S), jnp.float32),
    )(f2)


def _fin_body(m1_ref, m2_ref, a_ref, b_ref, q1c_ref, q2c_ref, w_ref,
              loss_ref, wl_ref):
    q1c = q1c_ref[...]
    q2c = q2c_ref[...]
    qm = jnp.abs(q1c - q2c)
    mean_q = jnp.sum(qm) * (1.0 / _B)
    margin_upper = 100.0 - (100.0 - mean_q) * _THRES
    margin_lower = mean_q * _THRES
    maskq = (qm < margin_lower) | (qm > margin_upper)
    q1z = jnp.where(maskq, 0.0, q1c)
    q2z = jnp.where(maskq, 0.0, q2c)
    f1h = q1z > q2z
    act = q1z != q2z
    count = jnp.sum(act.astype(jnp.float32))
    sum_q1 = jnp.sum(q1z)

    m1raw = m1_ref[...]
    m1 = jnp.concatenate([m1raw[:, 0:192], m1raw[:, 204:208]],
                         axis=1) * (1.0 / _C)
    m2 = m2_ref[...]
    d = m1 - m2
    mse_b = jnp.sum(d * d, axis=1, keepdims=True)
    mx1 = jnp.max(m1, axis=1, keepdims=True)
    mn1 = jnp.min(m1, axis=1, keepdims=True)
    mx2 = jnp.max(m2, axis=1, keepdims=True)
    mn2 = jnp.min(m2, axis=1, keepdims=True)
    fs_mx = jnp.where(f1h, mx2, mx1)
    fs_mn = jnp.where(f1h, mn2, mn1)
    mse_sum = jnp.sum(jnp.where(act, mse_b, 0.0))
    fs_max = jnp.max(jnp.where(act, fs_mx, -jnp.inf))
    fs_min = jnp.min(jnp.where(act, fs_mn, jnp.inf))

    a = a_ref[...]
    b = b_ref[...]
    dot = jnp.sum(a * b, axis=1, keepdims=True)
    na = jnp.sqrt(jnp.sum(a * a, axis=1, keepdims=True))
    nb = jnp.sqrt(jnp.sum(b * b, axis=1, keepdims=True))
    denom = jnp.maximum(na, _EPS) * jnp.maximum(nb, _EPS)
    cd = 1.0 - dot / denom
    cos_sum = jnp.sum(jnp.where(act, cd, 0.0))

    mse_loss = mse_sum / (count * jnp.float32(_S))
    ampify = 2.0 / (fs_max - fs_min)
    loss0 = ampify * mse_loss
    loss1 = cos_sum / count

    w0 = w_ref[0]
    w1 = w_ref[1]
    wl0 = loss0 * w0
    wl1 = loss1 * w1
    loss_all = wl0 + wl1

    zero_case = sum_q1 == 0.0
    loss_all = jnp.where(zero_case, 0.0, loss_all)
    wl0 = jnp.where(zero_case, 0.0, wl0)
    wl1 = jnp.where(zero_case, 0.0, wl1)

    loss_ref[...] = jnp.full((1, 1), loss_all)
    wl_ref[...] = jnp.concatenate(
        [jnp.full((1, 1), wl0), jnp.full((1, 1), wl1)], axis=1)


def _finalize(m1map, m2map, f1l1, f2l1, q1c, q2c, w):
    out = pl.pallas_call(
        _fin_body,
        in_specs=[
            pl.BlockSpec((_B, 256), lambda: (0, 0)),
            pl.BlockSpec((_B, _S), lambda: (0, 0)),
            pl.BlockSpec((_B, _D1), lambda: (0, 0)),
            pl.BlockSpec((_B, _D1), lambda: (0, 0)),
            pl.BlockSpec((_B, 1), lambda: (0, 0)),
            pl.BlockSpec((_B, 1), lambda: (0, 0)),
            pl.BlockSpec(memory_space=pltpu.SMEM),
        ],
        out_specs=[
            pl.BlockSpec((1, 1), lambda: (0, 0)),
            pl.BlockSpec((1, 2), lambda: (0, 0)),
        ],
        out_shape=[
            jax.ShapeDtypeStruct((1, 1), jnp.float32),
            jax.ShapeDtypeStruct((1, 2), jnp.float32),
        ],
    )(m1map, m2map, f1l1, f2l1, q1c, q2c, w)
    return out[0].reshape(()), out[1].reshape(2)


@jax.jit
def _qsd_loss(f1l0, f1l1, f2l0, f2l1, q1, q2, w):
    f1 = f1l0.reshape(_B, _C, _S)
    f2 = f2l0.reshape(_B, _C, _S)
    m1map = _sc_m1_map(f1)
    m2map = _m2_map(f2)
    return _finalize(m1map, m2map, f1l1, f2l1,
                     q1.reshape(_B, 1), q2.reshape(_B, 1), w)


def kernel(features_1_level0, features_1_level1, features_2_level0,
           features_2_level1, quality_1, quality_2, weights):
    return _qsd_loss(features_1_level0, features_1_level1,
                     features_2_level0, features_2_level1,
                     quality_1, quality_2, weights)
